# Initial kernel scaffold; baseline (speedup 1.0000x reference)
import jax
import jax.numpy as jnp
from jax.experimental import pallas as pl


def kernel(input_ids, embed, W0, W1):
    raise NotImplementedError("write your pallas kernel here")


# speedup vs baseline: 2.2914x; 2.2914x over previous
import functools

import jax
import jax.numpy as jnp
from jax import lax
from jax.experimental import pallas as pl
from jax.experimental.pallas import tpu as pltpu
from jax.experimental.pallas import tpu_sc as plsc

VOCAB = 100000
HIDDEN = 128
NC = 2
NS = 16
NW = NC * NS
B_TOTAL = 4096 * 50
B_PER_W = B_TOTAL // NW
CHUNK = 128
N_CHUNKS = B_PER_W // CHUNK
ROW_BLOCK = 2000
N_ROW_BLOCKS = VOCAB // ROW_BLOCK


def _transform_body(e_ref, w0_ref, w1_ref, t_ref):
    wc = lax.dot_general(
        w0_ref[...], w1_ref[...], (((0,), (1,)), ((), ())),
        preferred_element_type=jnp.float32)
    t_ref[...] = lax.dot_general(
        e_ref[...], wc, (((1,), (0,)), ((), ())),
        preferred_element_type=jnp.float32)


def _transform_table(embed, W0, W1):
    return pl.pallas_call(
        _transform_body,
        grid=(N_ROW_BLOCKS,),
        in_specs=[
            pl.BlockSpec((ROW_BLOCK, HIDDEN), lambda i: (i, 0)),
            pl.BlockSpec((HIDDEN, HIDDEN), lambda i: (0, 0)),
            pl.BlockSpec((HIDDEN, HIDDEN), lambda i: (0, 0)),
        ],
        out_specs=pl.BlockSpec((ROW_BLOCK, HIDDEN), lambda i: (i, 0)),
        out_shape=jax.ShapeDtypeStruct((VOCAB, HIDDEN), jnp.float32),
    )(embed, W0, W1)


def _gather_body(tab_hbm, idx_hbm, out_hbm, idx_v, rows_v, sem):
    wid = lax.axis_index("s") * NC + lax.axis_index("c")
    pltpu.sync_copy(idx_hbm.at[wid], idx_v)
    base = wid * B_PER_W

    def body(j, carry):
        pltpu.async_copy(tab_hbm.at[idx_v.at[j]], rows_v, sem).wait()
        pltpu.sync_copy(rows_v, out_hbm.at[pl.ds(base + j * CHUNK, CHUNK)])
        return carry

    lax.fori_loop(0, N_CHUNKS, body, 0)


def _gather_rows(table, idx):
    mesh = plsc.VectorSubcoreMesh(core_axis_name="c", subcore_axis_name="s")
    return pl.kernel(
        _gather_body,
        out_type=jax.ShapeDtypeStruct((B_TOTAL, HIDDEN), jnp.float32),
        mesh=mesh,
        scratch_types=[
            pltpu.VMEM((N_CHUNKS, CHUNK), jnp.int32),
            pltpu.VMEM((CHUNK, HIDDEN), jnp.float32),
            pltpu.SemaphoreType.DMA,
        ],
    )(table, idx)


def kernel(input_ids, embed, W0, W1):
    table = _transform_table(embed, W0, W1)
    idx = input_ids.reshape(NW, N_CHUNKS, CHUNK).astype(jnp.int32)
    out = _gather_rows(table, idx)
    return out.reshape(input_ids.shape[0], input_ids.shape[1], HIDDEN)
